# baseline (speedup 1.0000x reference)
import functools

import jax
import jax.numpy as jnp
from jax import lax
from jax.experimental import pallas as pl
from jax.experimental.pallas import tpu as pltpu
from jax.experimental.pallas import tpu_sc as plsc

N = 50000
M = 10000
K = 27
CIN = 256
COUT = 256

NC = 2
NS = 16
NW = NC * NS

SUB = 128

B = K * M
B_PAD = 270336

G = 8
GW = COUT // G
ACC_R = 50016
DUMMY = N
ZROWS = 1042
STRIPE = ACC_R // NS
DROWS = N // NS


@functools.cache
def _mesh():
    return plsc.VectorSubcoreMesh(
        core_axis_name="c", subcore_axis_name="s", num_cores=NC, num_subcores=NS
    )


@functools.cache
def _gather_fn(pad, width):
    subs_w = pad // NW // SUB
    rows_w = pad // NW

    def body(feat_hbm, idx_hbm, out_hbm, idx_v, rows_a, rows_b, rows_c,
             sem_a, sem_b, sem_c):
        c = lax.axis_index("c")
        s = lax.axis_index("s")
        wid = s * NC + c
        pltpu.sync_copy(idx_hbm.at[wid], idx_v)
        base = wid * rows_w

        def issue(i, buf, sem):
            j = jnp.minimum(i, subs_w - 1)
            pltpu.async_copy(feat_hbm.at[idx_v.at[j]], buf, sem)

        def drain(buf, sem):
            pltpu.make_async_copy(feat_hbm.at[idx_v.at[0]], buf, sem).wait()

        issue(0, rows_a, sem_a)
        issue(1, rows_b, sem_b)

        def triple(i3, carry):
            i = 3 * i3
            issue(i + 2, rows_c, sem_c)
            drain(rows_a, sem_a)
            pltpu.sync_copy(rows_a, out_hbm.at[pl.ds(base + i * SUB, SUB)])
            issue(i + 3, rows_a, sem_a)
            drain(rows_b, sem_b)
            pltpu.sync_copy(
                rows_b, out_hbm.at[pl.ds(base + (i + 1) * SUB, SUB)]
            )
            issue(i + 4, rows_b, sem_b)
            drain(rows_c, sem_c)
            pltpu.sync_copy(
                rows_c, out_hbm.at[pl.ds(base + (i + 2) * SUB, SUB)]
            )
            return carry

        lax.fori_loop(0, subs_w // 3, triple, 0)
        drain(rows_a, sem_a)
        drain(rows_b, sem_b)

    @jax.jit
    def run(features, idx3d):
        return pl.kernel(
            body,
            out_type=jax.ShapeDtypeStruct((pad, width), jnp.float32),
            mesh=_mesh(),
            scratch_types=[
                pltpu.VMEM((subs_w, SUB), jnp.int32),
                pltpu.VMEM((SUB, width), jnp.float32),
                pltpu.VMEM((SUB, width), jnp.float32),
                pltpu.VMEM((SUB, width), jnp.float32),
                pltpu.SemaphoreType.DMA,
                pltpu.SemaphoreType.DMA,
                pltpu.SemaphoreType.DMA,
            ],
        )(features, idx3d)

    return run


TM = 1000


def _gemm_body(x_ref, w_ref, o1_ref, o2_ref):
    res = jnp.dot(
        x_ref[...].astype(jnp.bfloat16),
        w_ref[0].astype(jnp.bfloat16),
        preferred_element_type=jnp.float32,
    )
    o1_ref[...] = res[:, :128]
    o2_ref[...] = res[:, 128:]


@functools.cache
def _gemm_fn(nk, pad):
    mt = M // TM

    @jax.jit
    def run(gathered, weight):
        return pl.pallas_call(
            _gemm_body,
            grid=(nk, mt),
            in_specs=[
                pl.BlockSpec((TM, CIN), lambda k, m: (k * mt + m, 0)),
                pl.BlockSpec((1, CIN, COUT), lambda k, m: (k, 0, 0)),
            ],
            out_specs=[
                pl.BlockSpec((TM, 128), lambda k, m: (k * mt + m, 0)),
                pl.BlockSpec((TM, 128), lambda k, m: (k * mt + m, 0)),
            ],
            out_shape=[
                jax.ShapeDtypeStruct((pad, 128), jnp.float32),
                jax.ShapeDtypeStruct((pad, 128), jnp.float32),
            ],
        )(gathered, weight)

    return run


ST = B_PAD // NS // SUB


def _scatter_body(p_lo, p_hi, oi_hbm, zeros_hbm, out_lo, out_hi,
                  idx_v, rows_a, rows_b, rows_c, acc, sem_a, sem_b, sem_c):
    c = lax.axis_index("c")
    s = lax.axis_index("s")
    pltpu.sync_copy(oi_hbm.at[s], idx_v)

    def accumulate(prod_hbm, idx_v, subs_t, rows_t, g32):
        def issue(i, buf, sem):
            r = jnp.minimum(i, subs_t - 1) * SUB + s * rows_t
            pltpu.async_copy(
                prod_hbm.at[pl.ds(r, SUB), pl.ds(g32, GW)], buf, sem
            )

        def drain(buf, sem):
            pltpu.make_async_copy(
                prod_hbm.at[pl.ds(0, SUB), pl.ds(g32, GW)], buf, sem
            ).wait()

        issue(0, rows_a, sem_a)
        issue(1, rows_b, sem_b)

        def triple(i3, carry):
            i = 3 * i3
            issue(i + 2, rows_c, sem_c)
            drain(rows_a, sem_a)
            pltpu.sync_copy(rows_a, acc.at[idx_v.at[i]], add=True)
            issue(i + 3, rows_a, sem_a)
            drain(rows_b, sem_b)
            pltpu.sync_copy(rows_b, acc.at[idx_v.at[i + 1]], add=True)
            issue(i + 4, rows_b, sem_b)
            drain(rows_c, sem_c)
            pltpu.sync_copy(rows_c, acc.at[idx_v.at[i + 2]], add=True)
            return carry

        lax.fori_loop(0, subs_t // 3, triple, 0)
        drain(rows_a, sem_a)
        drain(rows_b, sem_b)

    def run_group(prod, out_hbm, g32):
        for z in range(3):
            pltpu.sync_copy(
                zeros_hbm, acc.at[pl.ds(s * STRIPE + z * ZROWS, ZROWS)]
            )
        plsc.subcore_barrier()
        accumulate(prod, idx_v, ST, B_PAD // NS, g32)
        plsc.subcore_barrier()
        pltpu.sync_copy(
            acc.at[pl.ds(s * DROWS, DROWS)],
            out_hbm.at[pl.ds(s * DROWS, DROWS), pl.ds(g32, GW)],
        )
        plsc.subcore_barrier()

    for cc, (p_cc, o_cc) in enumerate(((p_lo, out_lo), (p_hi, out_hi))):
        @pl.when(c == cc)
        def _():
            for j in range(G // NC):
                run_group(p_cc, o_cc, j * GW)


@jax.jit
def _scatter(p_lo, p_hi, oi):
    zeros = jnp.zeros((ZROWS, GW), jnp.float32)
    return pl.kernel(
        _scatter_body,
        out_type=[
            jax.ShapeDtypeStruct((N, 128), jnp.float32),
            jax.ShapeDtypeStruct((N, 128), jnp.float32),
        ],
        mesh=_mesh(),
        scratch_types=[
            pltpu.VMEM((ST, SUB), jnp.int32),
            pltpu.VMEM((SUB, GW), jnp.float32),
            pltpu.VMEM((SUB, GW), jnp.float32),
            pltpu.VMEM((SUB, GW), jnp.float32),
            pltpu.VMEM_SHARED((ACC_R, GW), jnp.float32),
            pltpu.SemaphoreType.DMA,
            pltpu.SemaphoreType.DMA,
            pltpu.SemaphoreType.DMA,
        ],
        compiler_params=pltpu.CompilerParams(use_tc_tiling_on_sc=False),
    )(p_lo, p_hi, oi, zeros)


TN = 2000


def _bn_body(ylo_ref, yhi_ref, g_ref, b_ref, o_ref, s_ref, q_ref):
    p = pl.program_id(0)
    t = pl.program_id(1)

    @pl.when(p == 0)
    def _():
        @pl.when(t == 0)
        def _():
            s_ref[...] = jnp.zeros_like(s_ref)
            q_ref[...] = jnp.zeros_like(q_ref)

        x = jnp.concatenate([ylo_ref[...], yhi_ref[...]], axis=1)
        s_ref[...] += jnp.sum(x, axis=0, keepdims=True)
        q_ref[...] += jnp.sum(x * x, axis=0, keepdims=True)

    @pl.when(p == 1)
    def _():
        x = jnp.concatenate([ylo_ref[...], yhi_ref[...]], axis=1)
        mean = s_ref[...] * (1.0 / N)
        var = q_ref[...] * (1.0 / N) - mean * mean
        inv = lax.rsqrt(var + 1e-5) * g_ref[...]
        o_ref[...] = jnp.maximum((x - mean) * inv + b_ref[...], 0.0)


@jax.jit
def _bn(ylo, yhi, gamma2, beta2):
    return pl.pallas_call(
        _bn_body,
        grid=(2, N // TN),
        in_specs=[
            pl.BlockSpec((TN, 128), lambda p, t: (t, 0)),
            pl.BlockSpec((TN, 128), lambda p, t: (t, 0)),
            pl.BlockSpec((1, COUT), lambda p, t: (0, 0)),
            pl.BlockSpec((1, COUT), lambda p, t: (0, 0)),
        ],
        out_specs=pl.BlockSpec((TN, COUT), lambda p, t: (t, 0)),
        out_shape=jax.ShapeDtypeStruct((N, COUT), jnp.float32),
        scratch_shapes=[
            pltpu.VMEM((1, COUT), jnp.float32),
            pltpu.VMEM((1, COUT), jnp.float32),
        ],
    )(ylo, yhi, gamma2, beta2)


def _pad_idx(flat, pad_to, fill):
    return jnp.concatenate(
        [flat.astype(jnp.int32),
         jnp.full((pad_to - flat.shape[0],), fill, jnp.int32)]
    )


def kernel(features, in_idx, out_idx, weight, bias, gamma, beta):
    del bias
    ii = _pad_idx(in_idx.reshape(-1), B_PAD, 0).reshape(
        NW, B_PAD // NW // SUB, SUB
    )
    oi = _pad_idx(out_idx.reshape(-1), B_PAD, DUMMY).reshape(NS, ST, SUB)

    g = _gather_fn(B_PAD, CIN)(features, ii)
    p_lo, p_hi = _gemm_fn(K, B_PAD)(g, weight)
    ylo, yhi = _scatter(p_lo, p_hi, oi)
    return _bn(ylo, yhi, gamma.reshape(1, COUT), beta.reshape(1, COUT))


# speedup vs baseline: 1.0088x; 1.0088x over previous
import functools

import jax
import jax.numpy as jnp
from jax import lax
from jax.experimental import pallas as pl
from jax.experimental.pallas import tpu as pltpu
from jax.experimental.pallas import tpu_sc as plsc

N = 50000
M = 10000
K = 27
CIN = 256
COUT = 256

NC = 2
NS = 16
NW = NC * NS

SUB = 128

B = K * M
B_PAD = 270336

G = 8
GW = COUT // G
ACC_R = 50016
DUMMY = N
ZROWS = 1042
STRIPE = ACC_R // NS
DROWS = N // NS


@functools.cache
def _mesh():
    return plsc.VectorSubcoreMesh(
        core_axis_name="c", subcore_axis_name="s", num_cores=NC, num_subcores=NS
    )


@functools.cache
def _gather_fn(pad, width):
    subs_w = pad // NW // SUB
    rows_w = pad // NW

    def body(feat_hbm, idx_hbm, out_hbm, idx_v, rows_a, rows_b, sem_a,
             sem_b):
        c = lax.axis_index("c")
        s = lax.axis_index("s")
        wid = s * NC + c
        pltpu.sync_copy(idx_hbm.at[wid], idx_v)
        base = wid * rows_w

        def issue(i, buf, sem):
            j = jnp.minimum(i, subs_w - 1)
            pltpu.async_copy(feat_hbm.at[idx_v.at[j]], buf, sem)

        def drain(buf, sem):
            pltpu.make_async_copy(feat_hbm.at[idx_v.at[0]], buf, sem).wait()

        issue(0, rows_a, sem_a)

        def pair(i2, carry):
            i = 2 * i2
            issue(i + 1, rows_b, sem_b)
            drain(rows_a, sem_a)
            pltpu.sync_copy(rows_a, out_hbm.at[pl.ds(base + i * SUB, SUB)])
            issue(i + 2, rows_a, sem_a)
            drain(rows_b, sem_b)
            pltpu.sync_copy(
                rows_b, out_hbm.at[pl.ds(base + (i + 1) * SUB, SUB)]
            )
            return carry

        lax.fori_loop(0, subs_w // 2, pair, 0)
        drain(rows_a, sem_a)

    @jax.jit
    def run(features, idx3d):
        return pl.kernel(
            body,
            out_type=jax.ShapeDtypeStruct((pad, width), jnp.float32),
            mesh=_mesh(),
            scratch_types=[
                pltpu.VMEM((subs_w, SUB), jnp.int32),
                pltpu.VMEM((SUB, width), jnp.float32),
                pltpu.VMEM((SUB, width), jnp.float32),
                pltpu.SemaphoreType.DMA,
                pltpu.SemaphoreType.DMA,
            ],
        )(features, idx3d)

    return run


TM = 1000


def _gemm_body(x_ref, w_ref, o1_ref, o2_ref):
    res = jnp.dot(
        x_ref[...].astype(jnp.bfloat16),
        w_ref[0].astype(jnp.bfloat16),
        preferred_element_type=jnp.float32,
    )
    o1_ref[...] = res[:, :128]
    o2_ref[...] = res[:, 128:]


@functools.cache
def _gemm_fn(nk, pad):
    mt = M // TM

    @jax.jit
    def run(gathered, weight):
        return pl.pallas_call(
            _gemm_body,
            grid=(nk, mt),
            in_specs=[
                pl.BlockSpec((TM, CIN), lambda k, m: (k * mt + m, 0)),
                pl.BlockSpec((1, CIN, COUT), lambda k, m: (k, 0, 0)),
            ],
            out_specs=[
                pl.BlockSpec((TM, 128), lambda k, m: (k * mt + m, 0)),
                pl.BlockSpec((TM, 128), lambda k, m: (k * mt + m, 0)),
            ],
            out_shape=[
                jax.ShapeDtypeStruct((pad, 128), jnp.float32),
                jax.ShapeDtypeStruct((pad, 128), jnp.float32),
            ],
        )(gathered, weight)

    return run


ST = B_PAD // NS // SUB


def _scatter_body(p_lo, p_hi, oi_hbm, zeros_hbm, out_lo, out_hi,
                  idx_v, rows_a, rows_b, rows_c, acc, sem_a, sem_b, sem_c):
    c = lax.axis_index("c")
    s = lax.axis_index("s")
    pltpu.sync_copy(oi_hbm.at[s], idx_v)

    def accumulate(prod_hbm, idx_v, subs_t, rows_t, g32):
        def issue(i, buf, sem):
            r = jnp.minimum(i, subs_t - 1) * SUB + s * rows_t
            pltpu.async_copy(
                prod_hbm.at[pl.ds(r, SUB), pl.ds(g32, GW)], buf, sem
            )

        def drain(buf, sem):
            pltpu.make_async_copy(
                prod_hbm.at[pl.ds(0, SUB), pl.ds(g32, GW)], buf, sem
            ).wait()

        issue(0, rows_a, sem_a)
        issue(1, rows_b, sem_b)

        def triple(i3, carry):
            i = 3 * i3
            issue(i + 2, rows_c, sem_c)
            drain(rows_a, sem_a)
            pltpu.sync_copy(rows_a, acc.at[idx_v.at[i]], add=True)
            issue(i + 3, rows_a, sem_a)
            drain(rows_b, sem_b)
            pltpu.sync_copy(rows_b, acc.at[idx_v.at[i + 1]], add=True)
            issue(i + 4, rows_b, sem_b)
            drain(rows_c, sem_c)
            pltpu.sync_copy(rows_c, acc.at[idx_v.at[i + 2]], add=True)
            return carry

        lax.fori_loop(0, subs_t // 3, triple, 0)
        drain(rows_a, sem_a)
        drain(rows_b, sem_b)

    def run_group(prod, out_hbm, g32):
        for z in range(3):
            pltpu.sync_copy(
                zeros_hbm, acc.at[pl.ds(s * STRIPE + z * ZROWS, ZROWS)]
            )
        plsc.subcore_barrier()
        accumulate(prod, idx_v, ST, B_PAD // NS, g32)
        plsc.subcore_barrier()
        pltpu.sync_copy(
            acc.at[pl.ds(s * DROWS, DROWS)],
            out_hbm.at[pl.ds(s * DROWS, DROWS), pl.ds(g32, GW)],
        )
        plsc.subcore_barrier()

    for cc, (p_cc, o_cc) in enumerate(((p_lo, out_lo), (p_hi, out_hi))):
        @pl.when(c == cc)
        def _():
            for j in range(G // NC):
                run_group(p_cc, o_cc, j * GW)


@jax.jit
def _scatter(p_lo, p_hi, oi):
    zeros = jnp.zeros((ZROWS, GW), jnp.float32)
    return pl.kernel(
        _scatter_body,
        out_type=[
            jax.ShapeDtypeStruct((N, 128), jnp.float32),
            jax.ShapeDtypeStruct((N, 128), jnp.float32),
        ],
        mesh=_mesh(),
        scratch_types=[
            pltpu.VMEM((ST, SUB), jnp.int32),
            pltpu.VMEM((SUB, GW), jnp.float32),
            pltpu.VMEM((SUB, GW), jnp.float32),
            pltpu.VMEM((SUB, GW), jnp.float32),
            pltpu.VMEM_SHARED((ACC_R, GW), jnp.float32),
            pltpu.SemaphoreType.DMA,
            pltpu.SemaphoreType.DMA,
            pltpu.SemaphoreType.DMA,
        ],
        compiler_params=pltpu.CompilerParams(use_tc_tiling_on_sc=False),
    )(p_lo, p_hi, oi, zeros)


TN = 2000


def _bn_body(ylo_ref, yhi_ref, g_ref, b_ref, o_ref, s_ref, q_ref):
    p = pl.program_id(0)
    t = pl.program_id(1)

    @pl.when(p == 0)
    def _():
        @pl.when(t == 0)
        def _():
            s_ref[...] = jnp.zeros_like(s_ref)
            q_ref[...] = jnp.zeros_like(q_ref)

        x = jnp.concatenate([ylo_ref[...], yhi_ref[...]], axis=1)
        s_ref[...] += jnp.sum(x, axis=0, keepdims=True)
        q_ref[...] += jnp.sum(x * x, axis=0, keepdims=True)

    @pl.when(p == 1)
    def _():
        x = jnp.concatenate([ylo_ref[...], yhi_ref[...]], axis=1)
        mean = s_ref[...] * (1.0 / N)
        var = q_ref[...] * (1.0 / N) - mean * mean
        inv = lax.rsqrt(var + 1e-5) * g_ref[...]
        o_ref[...] = jnp.maximum((x - mean) * inv + b_ref[...], 0.0)


@jax.jit
def _bn(ylo, yhi, gamma2, beta2):
    return pl.pallas_call(
        _bn_body,
        grid=(2, N // TN),
        in_specs=[
            pl.BlockSpec((TN, 128), lambda p, t: (t, 0)),
            pl.BlockSpec((TN, 128), lambda p, t: (t, 0)),
            pl.BlockSpec((1, COUT), lambda p, t: (0, 0)),
            pl.BlockSpec((1, COUT), lambda p, t: (0, 0)),
        ],
        out_specs=pl.BlockSpec((TN, COUT), lambda p, t: (t, 0)),
        out_shape=jax.ShapeDtypeStruct((N, COUT), jnp.float32),
        scratch_shapes=[
            pltpu.VMEM((1, COUT), jnp.float32),
            pltpu.VMEM((1, COUT), jnp.float32),
        ],
    )(ylo, yhi, gamma2, beta2)


def _pad_idx(flat, pad_to, fill):
    return jnp.concatenate(
        [flat.astype(jnp.int32),
         jnp.full((pad_to - flat.shape[0],), fill, jnp.int32)]
    )


def kernel(features, in_idx, out_idx, weight, bias, gamma, beta):
    del bias
    ii = _pad_idx(in_idx.reshape(-1), B_PAD, 0).reshape(
        NW, B_PAD // NW // SUB, SUB
    )
    oi = _pad_idx(out_idx.reshape(-1), B_PAD, DUMMY).reshape(NS, ST, SUB)

    g = _gather_fn(B_PAD, CIN)(features, ii)
    p_lo, p_hi = _gemm_fn(K, B_PAD)(g, weight)
    ylo, yhi = _scatter(p_lo, p_hi, oi)
    return _bn(ylo, yhi, gamma.reshape(1, COUT), beta.reshape(1, COUT))
